# baseline (speedup 1.0000x reference)
import jax
import jax.numpy as jnp
from jax.experimental import pallas as pl

_F_BLK = 32


def _add_kernel(x_ref, emb_ref, o_ref):
    fe = emb_ref[...].T
    o_ref[...] = x_ref[...] + fe[None, :, :, None]


def _tc_call(x, emb_table):
    b, c, f, t = x.shape
    grid = (b, f // _F_BLK)
    return pl.pallas_call(
        _add_kernel,
        grid=grid,
        in_specs=[
            pl.BlockSpec((1, c, _F_BLK, t), lambda i, j: (i, 0, j, 0)),
            pl.BlockSpec((_F_BLK, c), lambda i, j: (j, 0)),
        ],
        out_specs=pl.BlockSpec((1, c, _F_BLK, t), lambda i, j: (i, 0, j, 0)),
        out_shape=jax.ShapeDtypeStruct(x.shape, x.dtype),
    )(x, emb_table)


def kernel(x, emb_table):
    b = x.shape[0]
    k = b // 2
    lo = _tc_call(x[:k], emb_table)
    hi = _tc_call(x[k:], emb_table)
    return jnp.concatenate([lo, hi], axis=0)


# speedup vs baseline: 3.0277x; 3.0277x over previous
import jax
import jax.numpy as jnp
from jax.experimental import pallas as pl

_F_BLK = 64


def _add_kernel(x_ref, emb_ref, o_ref):
    fe = emb_ref[...].T
    o_ref[...] = x_ref[...] + fe[None, :, :, None]


def kernel(x, emb_table):
    b, c, f, t = x.shape
    grid = (b, f // _F_BLK)
    return pl.pallas_call(
        _add_kernel,
        grid=grid,
        in_specs=[
            pl.BlockSpec((1, c, _F_BLK, t), lambda i, j: (i, 0, j, 0)),
            pl.BlockSpec((_F_BLK, c), lambda i, j: (j, 0)),
        ],
        out_specs=pl.BlockSpec((1, c, _F_BLK, t), lambda i, j: (i, 0, j, 0)),
        out_shape=jax.ShapeDtypeStruct(x.shape, x.dtype),
    )(x, emb_table)
